# baseline (speedup 1.0000x reference)
import functools

import jax
import jax.numpy as jnp
from jax import lax
from jax.experimental import pallas as pl
from jax.experimental.pallas import tpu as pltpu
from jax.experimental.pallas import tpu_sc as plsc

N = 10000
D = 128
N_PAD = 10240
DUMP = N
NW = 32
E_ALL = 320000 + N
ROWS_PT = N_PAD // 16
BLK = 512
NBLK = N_PAD // BLK
CD = 128
CHUNKS_D = 82
EPT_D = CHUNKS_D * CD
E_PAD_D = EPT_D * NW
CM = 128
CHUNKS_M = 84
HALF = CHUNKS_M // 2
JH = HALF // 2
EPT_M = CHUNKS_M * CM
E_PAD_M = EPT_M * NW

@functools.cache
def _mesh():
    return plsc.VectorSubcoreMesh(core_axis_name="c", subcore_axis_name="s",
                                  num_cores=2, num_subcores=16)


def _zero_slice(acc_sh, buf_v, base):
    def z(i, carry):
        for j in range(D // 16):
            buf_v[i, pl.ds(j * 16, 16)] = jnp.zeros((16,), jnp.float32)
        return carry
    lax.fori_loop(0, 64, z, 0)
    for r in range(ROWS_PT // 64):
        pltpu.sync_copy(buf_v.at[pl.ds(0, 64)],
                        acc_sh.at[pl.ds(base + r * 64, 64)])


def _copy_out(acc_sh, buf_v, out_hbm, c, base):
    for r in range(ROWS_PT // 64):
        pltpu.sync_copy(acc_sh.at[pl.ds(base + r * 64, 64)],
                        buf_v.at[pl.ds(0, 64)])
        pltpu.sync_copy(buf_v.at[pl.ds(0, 64)],
                        out_hbm.at[c, pl.ds(base + r * 64, 64)])


@functools.cache
def _build_deg_kernel():
    return functools.partial(
        pl.kernel,
        out_type=jax.ShapeDtypeStruct((2, N_PAD, D), jnp.float32),
        mesh=_mesh(),
        scratch_types=[
            pltpu.VMEM((CHUNKS_D + 1, CD), jnp.int32),
            pltpu.VMEM((CD, D), jnp.float32),
            pltpu.VMEM_SHARED((N_PAD, D), jnp.float32),
            pltpu.SemaphoreType.DMA,
            pltpu.SemaphoreType.DMA,
        ],
    )(_deg_body)


def _deg_body(dst_hbm, out_hbm, idx_all, buf_v, acc_sh, sem_a, sem_b):
    c = lax.axis_index("c")
    s = lax.axis_index("s")
    w = s * 2 + c
    base = s * ROWS_PT

    pltpu.sync_copy(dst_hbm.at[w], idx_all.at[pl.ds(0, CHUNKS_D)])
    for j in range(CD // 16):
        idx_all[CHUNKS_D, pl.ds(j * 16, 16)] = jnp.full((16,), DUMP, jnp.int32)

    _zero_slice(acc_sh, buf_v, base)
    plsc.subcore_barrier()

    def fill_ones(i, carry):
        for j in range(D // 16):
            buf_v[i, pl.ds(j * 16, 16)] = jnp.full((16,), 1.0, jnp.float32)
        return carry
    lax.fori_loop(0, CD, fill_ones, 0)

    pltpu.async_copy(buf_v, acc_sh.at[idx_all.at[0]], sem_a, add=True)

    def body(j, carry):
        i0 = 2 * j
        pltpu.async_copy(buf_v, acc_sh.at[idx_all.at[i0 + 1]], sem_b, add=True)
        pltpu.make_async_copy(buf_v, acc_sh.at[idx_all.at[i0]], sem_a).wait()
        pltpu.async_copy(buf_v, acc_sh.at[idx_all.at[i0 + 2]], sem_a, add=True)
        pltpu.make_async_copy(buf_v, acc_sh.at[idx_all.at[i0 + 1]], sem_b).wait()
        return carry
    lax.fori_loop(0, CHUNKS_D // 2, body, 0)
    pltpu.make_async_copy(buf_v, acc_sh.at[idx_all.at[CHUNKS_D]], sem_a).wait()
    plsc.subcore_barrier()

    _copy_out(acc_sh, buf_v, out_hbm, c, base)


@functools.cache
def _build_msg_kernel():
    return functools.partial(
        pl.kernel,
        out_type=jax.ShapeDtypeStruct((2, N_PAD, D), jnp.float32),
        mesh=_mesh(),
        scratch_types=[
            pltpu.VMEM((HALF + 1, 2, CM), jnp.int32),
            pltpu.VMEM((CM, D), jnp.float32),
            pltpu.VMEM((CM, D), jnp.float32),
            pltpu.VMEM_SHARED((N_PAD, D), jnp.float32),
            pltpu.SemaphoreType.DMA,
            pltpu.SemaphoreType.DMA,
        ],
    )(_msg_body)


def _msg_body(y_hbm, e2_hbm, out_hbm,
              idx_all, rows_a, rows_b, acc_sh, sem_ga, sem_gb):
    c = lax.axis_index("c")
    s = lax.axis_index("s")
    w = s * 2 + c
    base = s * ROWS_PT

    _zero_slice(acc_sh, rows_a, base)
    plsc.subcore_barrier()

    for h in range(2):
        pltpu.sync_copy(e2_hbm.at[w, pl.ds(h * HALF, HALF + 1)], idx_all)
        pltpu.async_copy(y_hbm.at[idx_all.at[0, 0]], rows_a, sem_ga)

        def body(j, carry):
            i0 = 2 * j
            pltpu.async_copy(y_hbm.at[idx_all.at[i0 + 1, 0]], rows_b, sem_gb)
            pltpu.make_async_copy(y_hbm.at[idx_all.at[i0, 0]], rows_a,
                                  sem_ga).wait()
            pltpu.sync_copy(rows_a, acc_sh.at[idx_all.at[i0, 1]], add=True)
            pltpu.async_copy(y_hbm.at[idx_all.at[i0 + 2, 0]], rows_a, sem_ga)
            pltpu.make_async_copy(y_hbm.at[idx_all.at[i0 + 1, 0]], rows_b,
                                  sem_gb).wait()
            pltpu.sync_copy(rows_b, acc_sh.at[idx_all.at[i0 + 1, 1]], add=True)
            return carry
        lax.fori_loop(0, JH, body, 0)
        pltpu.make_async_copy(y_hbm.at[idx_all.at[HALF, 0]], rows_a,
                              sem_ga).wait()
    plsc.subcore_barrier()

    _copy_out(acc_sh, rows_a, out_hbm, c, base)


def _pre_body(degp_ref, dinv_ref):
    x = degp_ref[...]
    d = x[0, :, 0:1] + x[1, :, 0:1]
    dinv_ref[...] = jnp.where(d > 0, lax.rsqrt(d), 0.0)


def _dinv(deg_partials):
    return pl.pallas_call(
        _pre_body,
        grid=(NBLK,),
        in_specs=[pl.BlockSpec((2, BLK, D), lambda i: (0, i, 0))],
        out_specs=pl.BlockSpec((BLK, 1), lambda i: (i, 0)),
        out_shape=jax.ShapeDtypeStruct((N_PAD, 1), jnp.float32),
    )(deg_partials)


def _first_body(x_ref, w_ref, d_ref, y_ref):
    xw = jnp.dot(x_ref[...], w_ref[...], preferred_element_type=jnp.float32)
    y_ref[...] = xw * d_ref[...]


def _first(x, W, dinv):
    return pl.pallas_call(
        _first_body,
        grid=(NBLK,),
        in_specs=[
            pl.BlockSpec((BLK, D), lambda i: (i, 0)),
            pl.BlockSpec((D, D), lambda i: (0, 0)),
            pl.BlockSpec((BLK, 1), lambda i: (i, 0)),
        ],
        out_specs=pl.BlockSpec((BLK, D), lambda i: (i, 0)),
        out_shape=jax.ShapeDtypeStruct((N_PAD, D), jnp.float32),
    )(x, W, dinv)


def _make_layer(has_res):
    def body(p_ref, d_ref, b_ref, *rest):
        if has_res:
            xprev_ref, w_ref, xnew_ref, y_ref = rest
        else:
            w_ref, xnew_ref, y_ref = rest
        d = d_ref[...]
        agg = (p_ref[0] + p_ref[1]) * d + b_ref[...]
        if has_res:
            agg = agg + xprev_ref[...]
        xn = jnp.maximum(agg, 0.0)
        xnew_ref[...] = xn
        y_ref[...] = jnp.dot(xn, w_ref[...],
                             preferred_element_type=jnp.float32) * d
    return body


def _layer(p, dinv, b, xprev, W):
    has_res = xprev is not None
    in_specs = [
        pl.BlockSpec((2, BLK, D), lambda i: (0, i, 0)),
        pl.BlockSpec((BLK, 1), lambda i: (i, 0)),
        pl.BlockSpec((1, D), lambda i: (0, 0)),
    ]
    args = [p, dinv, b]
    if has_res:
        in_specs.append(pl.BlockSpec((BLK, D), lambda i: (i, 0)))
        args.append(xprev)
    in_specs.append(pl.BlockSpec((D, D), lambda i: (0, 0)))
    args.append(W)
    return pl.pallas_call(
        _make_layer(has_res),
        grid=(NBLK,),
        in_specs=in_specs,
        out_specs=[
            pl.BlockSpec((BLK, D), lambda i: (i, 0)),
            pl.BlockSpec((BLK, D), lambda i: (i, 0)),
        ],
        out_shape=[
            jax.ShapeDtypeStruct((N_PAD, D), jnp.float32),
            jax.ShapeDtypeStruct((N_PAD, D), jnp.float32),
        ],
    )(*args)


def _final_body(p_ref, d_ref, b_ref, xprev_ref, wfc_ref, bfc_ref,
                out_ref, acc_ref):
    i = pl.program_id(0)
    d = d_ref[...]
    xn = jnp.maximum((p_ref[0] + p_ref[1]) * d + b_ref[...] + xprev_ref[...],
                     0.0)
    rows = i * BLK + lax.broadcasted_iota(jnp.int32, (BLK, D), 0)
    xn = jnp.where(rows < N, xn, 0.0)
    part = jnp.sum(xn, axis=0, keepdims=True)

    @pl.when(i == 0)
    def _():
        acc_ref[...] = jnp.zeros_like(acc_ref)

    acc_ref[0:1, :] += part

    @pl.when(i == NBLK - 1)
    def _():
        tot = acc_ref[0:1, :] * (1.0 / N)
        out_ref[...] = jnp.dot(tot, wfc_ref[...],
                               preferred_element_type=jnp.float32) + bfc_ref[...]


def _final(p, dinv, b, xprev, wfc_pad, bfc_pad):
    return pl.pallas_call(
        _final_body,
        grid=(NBLK,),
        in_specs=[
            pl.BlockSpec((2, BLK, D), lambda i: (0, i, 0)),
            pl.BlockSpec((BLK, 1), lambda i: (i, 0)),
            pl.BlockSpec((1, D), lambda i: (0, 0)),
            pl.BlockSpec((BLK, D), lambda i: (i, 0)),
            pl.BlockSpec((D, D), lambda i: (0, 0)),
            pl.BlockSpec((1, D), lambda i: (0, 0)),
        ],
        out_specs=pl.BlockSpec((1, D), lambda i: (0, 0)),
        out_shape=jax.ShapeDtypeStruct((1, D), jnp.float32),
        scratch_shapes=[pltpu.VMEM((8, D), jnp.float32)],
    )(p, dinv, b, xprev, wfc_pad, bfc_pad)


def _msg(y, e2):
    return _build_msg_kernel()(y, e2)


def kernel(node, edges, edges_attr, W0, b0, W1, b1, W2, b2, W3, b3, Wfc, bfc):
    del edges_attr
    loop = jnp.arange(N, dtype=edges.dtype)
    pad_d = jnp.full((E_PAD_D - E_ALL,), DUMP, dtype=edges.dtype)
    pad_m = jnp.full((E_PAD_M - E_ALL,), DUMP, dtype=edges.dtype)
    dst_d = jnp.concatenate([edges[1], loop, pad_d]).reshape(NW, CHUNKS_D, CD)
    src_all = jnp.concatenate([edges[0], loop])
    dst_all = jnp.concatenate([edges[1], loop])
    perm = jnp.argsort(src_all)
    src = jnp.concatenate([src_all[perm], pad_m]).reshape(NW, CHUNKS_M, 1, CM)
    dst = jnp.concatenate([dst_all[perm], pad_m]).reshape(NW, CHUNKS_M, 1, CM)
    tail = jnp.full((NW, 1, 2, CM), DUMP, dtype=edges.dtype)
    e2 = jnp.concatenate(
        [jnp.concatenate([src, dst], axis=2), tail], axis=1)

    node_p = jnp.pad(node, ((0, N_PAD - N), (0, 0)))
    wfc_pad = jnp.pad(Wfc, ((0, 0), (0, D - Wfc.shape[1])))
    bfc_pad = jnp.pad(bfc, (0, D - bfc.shape[0])).reshape(1, D)
    b0r = b0.reshape(1, D)
    b1r = b1.reshape(1, D)
    b2r = b2.reshape(1, D)
    b3r = b3.reshape(1, D)

    deg_p = _build_deg_kernel()(dst_d)
    dinv = _dinv(deg_p)

    y0 = _first(node_p, W0, dinv)
    p = _msg(y0, e2)
    x1, y1 = _layer(p, dinv, b0r, None, W1)
    p = _msg(y1, e2)
    x2, y2 = _layer(p, dinv, b1r, x1, W2)
    p = _msg(y2, e2)
    x3, y3 = _layer(p, dinv, b2r, x2, W3)
    p = _msg(y3, e2)
    out = _final(p, dinv, b3r, x3, wfc_pad, bfc_pad)
    return out[:, :2]


# speedup vs baseline: 3.9568x; 3.9568x over previous
import functools

import jax
import jax.numpy as jnp
from jax import lax
from jax.experimental import pallas as pl
from jax.experimental.pallas import tpu as pltpu
from jax.experimental.pallas import tpu_sc as plsc

N = 10000
D = 128
N_PAD = 10240
DUMP = N
NW = 32
E_ALL = 320000 + N
ROWS_PT = N_PAD // 16
BLK = 512
NBLK = N_PAD // BLK
CD = 128
CHUNKS_D = 82
EPT_D = CHUNKS_D * CD
E_PAD_D = EPT_D * NW
CM = 96
NSECT = 4
SECT = 27
CHUNKS_M = NSECT * SECT
RING = 3
GROUPS = SECT // RING
EPT_M = CHUNKS_M * CM
E_PAD_M = EPT_M * NW

@functools.cache
def _mesh():
    return plsc.VectorSubcoreMesh(core_axis_name="c", subcore_axis_name="s",
                                  num_cores=2, num_subcores=16)


def _zero_slice(acc_sh, buf_v, base):
    def z(i, carry):
        for j in range(D // 16):
            buf_v[i, pl.ds(j * 16, 16)] = jnp.zeros((16,), jnp.float32)
        return carry
    lax.fori_loop(0, 64, z, 0)
    for r in range(ROWS_PT // 64):
        pltpu.sync_copy(buf_v.at[pl.ds(0, 64)],
                        acc_sh.at[pl.ds(base + r * 64, 64)])


def _copy_out(acc_sh, buf_v, out_hbm, c, base):
    for r in range(ROWS_PT // 64):
        pltpu.sync_copy(acc_sh.at[pl.ds(base + r * 64, 64)],
                        buf_v.at[pl.ds(0, 64)])
        pltpu.sync_copy(buf_v.at[pl.ds(0, 64)],
                        out_hbm.at[c, pl.ds(base + r * 64, 64)])


@functools.cache
def _build_deg_kernel():
    return functools.partial(
        pl.kernel,
        out_type=jax.ShapeDtypeStruct((2, N_PAD, D), jnp.float32),
        mesh=_mesh(),
        scratch_types=[
            pltpu.VMEM((CHUNKS_D + 1, CD), jnp.int32),
            pltpu.VMEM((CD, D), jnp.float32),
            pltpu.VMEM_SHARED((N_PAD, D), jnp.float32),
            pltpu.SemaphoreType.DMA,
            pltpu.SemaphoreType.DMA,
        ],
    )(_deg_body)


def _deg_body(dst_hbm, out_hbm, idx_all, buf_v, acc_sh, sem_a, sem_b):
    c = lax.axis_index("c")
    s = lax.axis_index("s")
    w = s * 2 + c
    base = s * ROWS_PT

    pltpu.sync_copy(dst_hbm.at[w], idx_all.at[pl.ds(0, CHUNKS_D)])
    for j in range(CD // 16):
        idx_all[CHUNKS_D, pl.ds(j * 16, 16)] = jnp.full((16,), DUMP, jnp.int32)

    _zero_slice(acc_sh, buf_v, base)
    plsc.subcore_barrier()

    def fill_ones(i, carry):
        for j in range(D // 16):
            buf_v[i, pl.ds(j * 16, 16)] = jnp.full((16,), 1.0, jnp.float32)
        return carry
    lax.fori_loop(0, CD, fill_ones, 0)

    pltpu.async_copy(buf_v, acc_sh.at[idx_all.at[0]], sem_a, add=True)

    def body(j, carry):
        i0 = 2 * j
        pltpu.async_copy(buf_v, acc_sh.at[idx_all.at[i0 + 1]], sem_b, add=True)
        pltpu.make_async_copy(buf_v, acc_sh.at[idx_all.at[i0]], sem_a).wait()
        pltpu.async_copy(buf_v, acc_sh.at[idx_all.at[i0 + 2]], sem_a, add=True)
        pltpu.make_async_copy(buf_v, acc_sh.at[idx_all.at[i0 + 1]], sem_b).wait()
        return carry
    lax.fori_loop(0, CHUNKS_D // 2, body, 0)
    pltpu.make_async_copy(buf_v, acc_sh.at[idx_all.at[CHUNKS_D]], sem_a).wait()
    plsc.subcore_barrier()

    _copy_out(acc_sh, buf_v, out_hbm, c, base)


@functools.cache
def _build_msg_kernel():
    return functools.partial(
        pl.kernel,
        out_type=jax.ShapeDtypeStruct((2, N_PAD, D), jnp.float32),
        mesh=_mesh(),
        scratch_types=[
            pltpu.VMEM((SECT, 2, CM), jnp.int32),
            pltpu.VMEM((CM, D), jnp.float32),
            pltpu.VMEM((CM, D), jnp.float32),
            pltpu.VMEM((CM, D), jnp.float32),
            pltpu.VMEM_SHARED((N_PAD, D), jnp.float32),
            pltpu.SemaphoreType.DMA,
            pltpu.SemaphoreType.DMA,
            pltpu.SemaphoreType.DMA,
        ],
    )(_msg_body)


def _msg_body(y_hbm, e2_hbm, out_hbm,
              idx_all, rows_0, rows_1, rows_2, acc_sh, sem_0, sem_1, sem_2):
    c = lax.axis_index("c")
    s = lax.axis_index("s")
    w = s * 2 + c
    base = s * ROWS_PT
    rows = (rows_0, rows_1, rows_2)
    sems = (sem_0, sem_1, sem_2)

    _zero_slice(acc_sh, rows_0, base)
    plsc.subcore_barrier()

    for h in range(NSECT):
        pltpu.sync_copy(e2_hbm.at[w, pl.ds(h * SECT, SECT)], idx_all)
        for r in range(RING):
            pltpu.async_copy(y_hbm.at[idx_all.at[r, 0]], rows[r], sems[r])

        def body(j, carry):
            i0 = RING * j
            for r in range(RING):
                pltpu.make_async_copy(y_hbm.at[idx_all.at[i0 + r, 0]],
                                      rows[r], sems[r]).wait()
                pltpu.sync_copy(rows[r], acc_sh.at[idx_all.at[i0 + r, 1]],
                                add=True)
                pltpu.async_copy(y_hbm.at[idx_all.at[i0 + r + RING, 0]],
                                 rows[r], sems[r])
            return carry
        lax.fori_loop(0, GROUPS - 1, body, 0)
        i0 = RING * (GROUPS - 1)
        for r in range(RING):
            pltpu.make_async_copy(y_hbm.at[idx_all.at[i0 + r, 0]],
                                  rows[r], sems[r]).wait()
            pltpu.sync_copy(rows[r], acc_sh.at[idx_all.at[i0 + r, 1]],
                            add=True)
    plsc.subcore_barrier()

    _copy_out(acc_sh, rows_0, out_hbm, c, base)


def _pre_body(degp_ref, dinv_ref):
    x = degp_ref[...]
    d = x[0, :, 0:1] + x[1, :, 0:1]
    dinv_ref[...] = jnp.where(d > 0, lax.rsqrt(d), 0.0)


def _dinv(deg_partials):
    return pl.pallas_call(
        _pre_body,
        grid=(NBLK,),
        in_specs=[pl.BlockSpec((2, BLK, D), lambda i: (0, i, 0))],
        out_specs=pl.BlockSpec((BLK, 1), lambda i: (i, 0)),
        out_shape=jax.ShapeDtypeStruct((N_PAD, 1), jnp.float32),
    )(deg_partials)


def _first_body(x_ref, w_ref, d_ref, y_ref):
    xw = jnp.dot(x_ref[...], w_ref[...], preferred_element_type=jnp.float32)
    y_ref[...] = xw * d_ref[...]


def _first(x, W, dinv):
    return pl.pallas_call(
        _first_body,
        grid=(NBLK,),
        in_specs=[
            pl.BlockSpec((BLK, D), lambda i: (i, 0)),
            pl.BlockSpec((D, D), lambda i: (0, 0)),
            pl.BlockSpec((BLK, 1), lambda i: (i, 0)),
        ],
        out_specs=pl.BlockSpec((BLK, D), lambda i: (i, 0)),
        out_shape=jax.ShapeDtypeStruct((N_PAD, D), jnp.float32),
    )(x, W, dinv)


def _make_layer(has_res):
    def body(p_ref, d_ref, b_ref, *rest):
        if has_res:
            xprev_ref, w_ref, xnew_ref, y_ref = rest
        else:
            w_ref, xnew_ref, y_ref = rest
        d = d_ref[...]
        agg = (p_ref[0] + p_ref[1]) * d + b_ref[...]
        if has_res:
            agg = agg + xprev_ref[...]
        xn = jnp.maximum(agg, 0.0)
        xnew_ref[...] = xn
        y_ref[...] = jnp.dot(xn, w_ref[...],
                             preferred_element_type=jnp.float32) * d
    return body


def _layer(p, dinv, b, xprev, W):
    has_res = xprev is not None
    in_specs = [
        pl.BlockSpec((2, BLK, D), lambda i: (0, i, 0)),
        pl.BlockSpec((BLK, 1), lambda i: (i, 0)),
        pl.BlockSpec((1, D), lambda i: (0, 0)),
    ]
    args = [p, dinv, b]
    if has_res:
        in_specs.append(pl.BlockSpec((BLK, D), lambda i: (i, 0)))
        args.append(xprev)
    in_specs.append(pl.BlockSpec((D, D), lambda i: (0, 0)))
    args.append(W)
    return pl.pallas_call(
        _make_layer(has_res),
        grid=(NBLK,),
        in_specs=in_specs,
        out_specs=[
            pl.BlockSpec((BLK, D), lambda i: (i, 0)),
            pl.BlockSpec((BLK, D), lambda i: (i, 0)),
        ],
        out_shape=[
            jax.ShapeDtypeStruct((N_PAD, D), jnp.float32),
            jax.ShapeDtypeStruct((N_PAD, D), jnp.float32),
        ],
    )(*args)


def _final_body(p_ref, d_ref, b_ref, xprev_ref, wfc_ref, bfc_ref,
                out_ref, acc_ref):
    i = pl.program_id(0)
    d = d_ref[...]
    xn = jnp.maximum((p_ref[0] + p_ref[1]) * d + b_ref[...] + xprev_ref[...],
                     0.0)
    rows = i * BLK + lax.broadcasted_iota(jnp.int32, (BLK, D), 0)
    xn = jnp.where(rows < N, xn, 0.0)
    part = jnp.sum(xn, axis=0, keepdims=True)

    @pl.when(i == 0)
    def _():
        acc_ref[...] = jnp.zeros_like(acc_ref)

    acc_ref[0:1, :] += part

    @pl.when(i == NBLK - 1)
    def _():
        tot = acc_ref[0:1, :] * (1.0 / N)
        out_ref[...] = jnp.dot(tot, wfc_ref[...],
                               preferred_element_type=jnp.float32) + bfc_ref[...]


def _final(p, dinv, b, xprev, wfc_pad, bfc_pad):
    return pl.pallas_call(
        _final_body,
        grid=(NBLK,),
        in_specs=[
            pl.BlockSpec((2, BLK, D), lambda i: (0, i, 0)),
            pl.BlockSpec((BLK, 1), lambda i: (i, 0)),
            pl.BlockSpec((1, D), lambda i: (0, 0)),
            pl.BlockSpec((BLK, D), lambda i: (i, 0)),
            pl.BlockSpec((D, D), lambda i: (0, 0)),
            pl.BlockSpec((1, D), lambda i: (0, 0)),
        ],
        out_specs=pl.BlockSpec((1, D), lambda i: (0, 0)),
        out_shape=jax.ShapeDtypeStruct((1, D), jnp.float32),
        scratch_shapes=[pltpu.VMEM((8, D), jnp.float32)],
    )(p, dinv, b, xprev, wfc_pad, bfc_pad)


def _msg(y, e2):
    return _build_msg_kernel()(y, e2)


def kernel(node, edges, edges_attr, W0, b0, W1, b1, W2, b2, W3, b3, Wfc, bfc):
    del edges_attr
    loop = jnp.arange(N, dtype=edges.dtype)
    pad_d = jnp.full((E_PAD_D - E_ALL,), DUMP, dtype=edges.dtype)
    pad_m = jnp.full((E_PAD_M - E_ALL,), DUMP, dtype=edges.dtype)
    dst_d = jnp.concatenate([edges[1], loop, pad_d]).reshape(NW, CHUNKS_D, CD)
    src = jnp.concatenate([edges[0], loop, pad_m]).reshape(NW, CHUNKS_M, 1, CM)
    dst = jnp.concatenate([edges[1], loop, pad_m]).reshape(NW, CHUNKS_M, 1, CM)
    e2 = jnp.concatenate([src, dst], axis=2)

    node_p = jnp.pad(node, ((0, N_PAD - N), (0, 0)))
    wfc_pad = jnp.pad(Wfc, ((0, 0), (0, D - Wfc.shape[1])))
    bfc_pad = jnp.pad(bfc, (0, D - bfc.shape[0])).reshape(1, D)
    b0r = b0.reshape(1, D)
    b1r = b1.reshape(1, D)
    b2r = b2.reshape(1, D)
    b3r = b3.reshape(1, D)

    deg_p = _build_deg_kernel()(dst_d)
    dinv = _dinv(deg_p)

    y0 = _first(node_p, W0, dinv)
    p = _msg(y0, e2)
    x1, y1 = _layer(p, dinv, b0r, None, W1)
    p = _msg(y1, e2)
    x2, y2 = _layer(p, dinv, b1r, x1, W2)
    p = _msg(y2, e2)
    x3, y3 = _layer(p, dinv, b2r, x2, W3)
    p = _msg(y3, e2)
    out = _final(p, dinv, b3r, x3, wfc_pad, bfc_pad)
    return out[:, :2]
